# baseline (speedup 1.0000x reference)
import functools

import jax
import jax.numpy as jnp
from jax import lax
from jax.experimental import pallas as pl
from jax.experimental.pallas import tpu as pltpu
from jax.experimental.pallas import tpu_sc as plsc

VOCAB1 = 1000001
EMBED_DIM = 64
BATCH = 4096
SEQ = 200

NC = 2
NS = 16
NW = NC * NS

TOTAL = BATCH * SEQ
B_PER_W = TOTAL // NW
CHUNK = 128
N_CHUNKS = B_PER_W // CHUNK


def _build_kernel():
    mesh = plsc.VectorSubcoreMesh(core_axis_name="c", subcore_axis_name="s")

    @functools.partial(
        pl.kernel,
        mesh=mesh,
        out_type=jax.ShapeDtypeStruct((NW, N_CHUNKS, CHUNK, EMBED_DIM), jnp.float32),
        compiler_params=pltpu.CompilerParams(use_tc_tiling_on_sc=False),
        scratch_types=[
            pltpu.VMEM((N_CHUNKS, CHUNK), jnp.int32),
            pltpu.VMEM((CHUNK, EMBED_DIM), jnp.float32),
            pltpu.SemaphoreType.DMA,
        ],
    )
    def k(table_hbm, ids_hbm, out_hbm, idx_v, rows_v, gsem):
        wid = lax.axis_index("s") * NC + lax.axis_index("c")
        pltpu.sync_copy(ids_hbm.at[wid], idx_v)

        def body(c, carry):
            pltpu.async_copy(table_hbm.at[idx_v.at[c]], rows_v, gsem).wait()
            pltpu.sync_copy(rows_v, out_hbm.at[wid, c])
            return carry

        lax.fori_loop(0, N_CHUNKS, body, 0)

    return k


_k = _build_kernel()


@jax.jit
def kernel(input_ids, table):
    ids = input_ids.reshape(NW, N_CHUNKS, CHUNK).astype(jnp.int32)
    out = _k(table, ids)
    return out.reshape(BATCH, SEQ, EMBED_DIM)


# speedup vs baseline: 1.1137x; 1.1137x over previous
import functools

import jax
import jax.numpy as jnp
from jax import lax
from jax.experimental import pallas as pl
from jax.experimental.pallas import tpu as pltpu
from jax.experimental.pallas import tpu_sc as plsc

VOCAB1 = 1000001
EMBED_DIM = 64
BATCH = 4096
SEQ = 200

NC = 2
NS = 16
NW = NC * NS

TOTAL = BATCH * SEQ
B_PER_W = TOTAL // NW
CHUNK = 128
N_CHUNKS = B_PER_W // CHUNK
K = 4
NSETS = 2
N_GROUPS = N_CHUNKS // K
N_ITERS = N_GROUPS // NSETS


def _build_kernel():
    mesh = plsc.VectorSubcoreMesh(core_axis_name="c", subcore_axis_name="s")

    @functools.partial(
        pl.kernel,
        mesh=mesh,
        out_type=jax.ShapeDtypeStruct((NW, N_CHUNKS, CHUNK, EMBED_DIM), jnp.float32),
        compiler_params=pltpu.CompilerParams(use_tc_tiling_on_sc=False),
        scratch_types=[
            pltpu.VMEM((N_CHUNKS, CHUNK), jnp.int32),
            pltpu.VMEM((NSETS, K, CHUNK, EMBED_DIM), jnp.float32),
            pltpu.SemaphoreType.DMA,
            pltpu.SemaphoreType.DMA,
            pltpu.SemaphoreType.DMA,
        ],
    )
    def k(table_hbm, ids_hbm, out_hbm, idx_v, rows_v, gsem, ssem0, ssem1):
        wid = lax.axis_index("s") * NC + lax.axis_index("c")
        pltpu.sync_copy(ids_hbm.at[wid], idx_v)
        ssems = (ssem0, ssem1)

        def body(i, carry):
            for p in range(NSETS):
                g = i * NSETS + p
                @pl.when(i >= 1)
                def _():
                    for b in range(K):
                        pltpu.make_async_copy(
                            rows_v.at[p, b], out_hbm.at[wid, b], ssems[p]
                        ).wait()
                for b in range(K):
                    pltpu.async_copy(
                        table_hbm.at[idx_v.at[g * K + b]], rows_v.at[p, b], gsem
                    )
                for b in range(K):
                    pltpu.make_async_copy(
                        table_hbm.at[idx_v.at[g * K + b]], rows_v.at[p, b], gsem
                    ).wait()
                for b in range(K):
                    pltpu.async_copy(
                        rows_v.at[p, b], out_hbm.at[wid, g * K + b], ssems[p]
                    )
            return carry

        lax.fori_loop(0, N_ITERS, body, 0)
        for p in range(NSETS):
            for b in range(K):
                pltpu.make_async_copy(
                    rows_v.at[p, b], out_hbm.at[wid, b], ssems[p]
                ).wait()

    return k


_k = _build_kernel()


@jax.jit
def kernel(input_ids, table):
    ids = input_ids.reshape(NW, N_CHUNKS, CHUNK).astype(jnp.int32)
    out = _k(table, ids)
    return out.reshape(BATCH, SEQ, EMBED_DIM)


# speedup vs baseline: 1.1149x; 1.0011x over previous
import functools

import jax
import jax.numpy as jnp
from jax import lax
from jax.experimental import pallas as pl
from jax.experimental.pallas import tpu as pltpu
from jax.experimental.pallas import tpu_sc as plsc

VOCAB1 = 1000001
EMBED_DIM = 64
BATCH = 4096
SEQ = 200

NC = 2
NS = 16
NW = NC * NS

TOTAL = BATCH * SEQ
B_PER_W = TOTAL // NW
CHUNK = 512
N_CHUNKS = B_PER_W // CHUNK
K = 1
NSETS = 2
N_GROUPS = N_CHUNKS // K
N_ITERS = N_GROUPS // NSETS


def _build_kernel():
    mesh = plsc.VectorSubcoreMesh(core_axis_name="c", subcore_axis_name="s")

    @functools.partial(
        pl.kernel,
        mesh=mesh,
        out_type=jax.ShapeDtypeStruct((NW, N_CHUNKS, CHUNK, EMBED_DIM), jnp.float32),
        compiler_params=pltpu.CompilerParams(use_tc_tiling_on_sc=False),
        scratch_types=[
            pltpu.VMEM((N_CHUNKS, CHUNK), jnp.int32),
            pltpu.VMEM((NSETS, K, CHUNK, EMBED_DIM), jnp.float32),
            pltpu.SemaphoreType.DMA,
            pltpu.SemaphoreType.DMA,
            pltpu.SemaphoreType.DMA,
        ],
    )
    def k(table_hbm, ids_hbm, out_hbm, idx_v, rows_v, gsem, ssem0, ssem1):
        wid = lax.axis_index("s") * NC + lax.axis_index("c")
        pltpu.sync_copy(ids_hbm.at[wid], idx_v)
        ssems = (ssem0, ssem1)

        def body(i, carry):
            for p in range(NSETS):
                g = i * NSETS + p
                @pl.when(i >= 1)
                def _():
                    for b in range(K):
                        pltpu.make_async_copy(
                            rows_v.at[p, b], out_hbm.at[wid, b], ssems[p]
                        ).wait()
                for b in range(K):
                    pltpu.async_copy(
                        table_hbm.at[idx_v.at[g * K + b]], rows_v.at[p, b], gsem
                    )
                for b in range(K):
                    pltpu.make_async_copy(
                        table_hbm.at[idx_v.at[g * K + b]], rows_v.at[p, b], gsem
                    ).wait()
                for b in range(K):
                    pltpu.async_copy(
                        rows_v.at[p, b], out_hbm.at[wid, g * K + b], ssems[p]
                    )
            return carry

        lax.fori_loop(0, N_ITERS, body, 0)
        for p in range(NSETS):
            for b in range(K):
                pltpu.make_async_copy(
                    rows_v.at[p, b], out_hbm.at[wid, b], ssems[p]
                ).wait()

    return k


_k = _build_kernel()


@jax.jit
def kernel(input_ids, table):
    ids = input_ids.reshape(NW, N_CHUNKS, CHUNK).astype(jnp.int32)
    out = _k(table, ids)
    return out.reshape(BATCH, SEQ, EMBED_DIM)


# speedup vs baseline: 1.1157x; 1.0007x over previous
import functools

import jax
import jax.numpy as jnp
from jax import lax
from jax.experimental import pallas as pl
from jax.experimental.pallas import tpu as pltpu
from jax.experimental.pallas import tpu_sc as plsc

VOCAB1 = 1000001
EMBED_DIM = 64
BATCH = 4096
SEQ = 200

NC = 2
NS = 16
NW = NC * NS

TOTAL = BATCH * SEQ
B_PER_W = TOTAL // NW
CHUNK = 256
N_CHUNKS = B_PER_W // CHUNK
R = 4
LA = 2
N_ITERS = N_CHUNKS // R


def _build_kernel():
    mesh = plsc.VectorSubcoreMesh(core_axis_name="c", subcore_axis_name="s")

    @functools.partial(
        pl.kernel,
        mesh=mesh,
        out_type=jax.ShapeDtypeStruct((NW, N_CHUNKS, CHUNK, EMBED_DIM), jnp.float32),
        compiler_params=pltpu.CompilerParams(use_tc_tiling_on_sc=False),
        scratch_types=[
            pltpu.VMEM((N_CHUNKS, CHUNK), jnp.int32),
            pltpu.VMEM((R, CHUNK, EMBED_DIM), jnp.float32),
        ]
        + [pltpu.SemaphoreType.DMA] * (2 * R),
    )
    def k(table_hbm, ids_hbm, out_hbm, idx_v, rows_v, *sems):
        gsems = sems[:R]
        ssems = sems[R:]
        wid = lax.axis_index("s") * NC + lax.axis_index("c")
        pltpu.sync_copy(ids_hbm.at[wid], idx_v)

        def fire_gather(g, s):
            pltpu.async_copy(table_hbm.at[idx_v.at[g]], rows_v.at[s], gsems[s])

        def wait_gather(g, s):
            pltpu.make_async_copy(
                table_hbm.at[idx_v.at[g]], rows_v.at[s], gsems[s]
            ).wait()

        def fire_store(g, s):
            pltpu.async_copy(rows_v.at[s], out_hbm.at[wid, g], ssems[s])

        def drain_store(s):
            pltpu.make_async_copy(rows_v.at[s], out_hbm.at[wid, 0], ssems[s]).wait()

        for g0 in range(LA):
            fire_gather(g0, g0)
        for p in range(R):
            sf = (p + LA) % R
            if p >= R - LA:
                drain_store(sf)
            fire_gather(p + LA, sf)
            wait_gather(p, p)
            fire_store(p, p)

        def body(i, carry):
            for p in range(R):
                g = i * R + p
                sf = (p + LA) % R
                drain_store(sf)
                fire_gather(g + LA, sf)
                wait_gather(g, p)
                fire_store(g, p)
            return carry

        lax.fori_loop(1, N_ITERS - 1, body, 0)

        base = (N_ITERS - 1) * R
        for p in range(R):
            g = base + p
            sf = (p + LA) % R
            drain_store(sf)
            if p < R - LA:
                fire_gather(g + LA, sf)
            wait_gather(g, p)
            fire_store(g, p)
        for p in range(R - LA, R):
            drain_store(p)

    return k


_k = _build_kernel()


@jax.jit
def kernel(input_ids, table):
    ids = input_ids.reshape(NW, N_CHUNKS, CHUNK).astype(jnp.int32)
    out = _k(table, ids)
    return out.reshape(BATCH, SEQ, EMBED_DIM)
